# baseline (speedup 1.0000x reference)
import jax
import jax.numpy as jnp
from jax.experimental import pallas as pl
from jax.experimental.pallas import tpu as pltpu

_NUM_MODULES = 11
_HIDDEN = 256
_B = 64
_NEG = -1e30


def _fused_body(xa_ref, xb_ref, w_ref, bp_ref, m_ref, bt_ref, out_ref,
                acc, runmax):
    i = pl.program_id(0)
    nt = pl.num_programs(0)
    t = xa_ref.shape[0]
    w = w_ref[...]

    @pl.when(i == 0)
    def _():
        acc[...] = jnp.zeros((_B, 2 * _NUM_MODULES), jnp.float32)
        runmax[...] = jnp.full((1, _NUM_MODULES), _NEG, jnp.float32)

    for k, xr in enumerate((xa_ref, xb_ref)):
        x = xr[...]
        s = jnp.dot(x, w, preferred_element_type=jnp.float32)
        m = m_ref[k * t:(k + 1) * t].astype(jnp.int32)
        iota = jax.lax.broadcasted_iota(jnp.int32, (1, _NUM_MODULES), 1)
        sa = jnp.where(m == iota, s[:, :_NUM_MODULES], _NEG)
        tmax = jnp.max(sa, axis=0, keepdims=True)

        old_raw = runmax[...]
        new_raw = jnp.maximum(old_raw, tmax)
        runmax[...] = new_raw
        stab_old = jnp.where(old_raw < -1e29, 0.0, old_raw)
        stab_new = jnp.where(new_raw < -1e29, 0.0, new_raw)
        factor = jnp.exp(stab_old - stab_new)

        mex = jnp.exp(sa - stab_new)
        cat = jnp.concatenate(
            [mex, mex * s[:, _NUM_MODULES:2 * _NUM_MODULES]], axis=1)
        bt = bt_ref[0, :, k * t:(k + 1) * t]
        io64 = jax.lax.broadcasted_iota(jnp.int32, (_B, t), 0)
        ohbt = (io64 == bt).astype(jnp.float32)
        contrib = jnp.dot(ohbt, cat, preferred_element_type=jnp.float32)

        facc = jnp.concatenate([factor, factor], axis=1)
        acc[...] = acc[...] * facc + contrib

    @pl.when(i == nt - 1)
    def _():
        den = acc[:, :_NUM_MODULES]
        num = acc[:, _NUM_MODULES:2 * _NUM_MODULES]
        out_ref[...] = (num + bp_ref[...] * den) / (den + 1e-16)


def kernel(x, Wa, ba, Wp, bp, module_assign, batch):
    n = x.shape[0]
    t = 10000
    nt = n // (2 * t)
    wcat = jnp.concatenate([Wa, Wp], axis=0).T
    bp_row = bp[None, :]
    m_col = module_assign.astype(jnp.int8).reshape(n, 1)
    b_row = batch.astype(jnp.int32).reshape(nt, 1, 2 * t)

    out = pl.pallas_call(
        _fused_body,
        grid=(nt,),
        in_specs=[
            pl.BlockSpec((t, _HIDDEN), lambda i: (2 * i, 0)),
            pl.BlockSpec((t, _HIDDEN), lambda i: (2 * i + 1, 0)),
            pl.BlockSpec((_HIDDEN, 2 * _NUM_MODULES), lambda i: (0, 0)),
            pl.BlockSpec((1, _NUM_MODULES), lambda i: (0, 0)),
            pl.BlockSpec((2 * t, 1), lambda i: (i, 0)),
            pl.BlockSpec((1, 1, 2 * t), lambda i: (i, 0, 0)),
        ],
        out_specs=pl.BlockSpec((_B, _NUM_MODULES), lambda i: (0, 0)),
        out_shape=jax.ShapeDtypeStruct((_B, _NUM_MODULES), jnp.float32),
        scratch_shapes=[
            pltpu.VMEM((_B, 2 * _NUM_MODULES), jnp.float32),
            pltpu.VMEM((1, _NUM_MODULES), jnp.float32),
        ],
        compiler_params=pltpu.CompilerParams(
            dimension_semantics=("arbitrary",)),
    )(x, x, wcat, bp_row, m_col, b_row)

    return out


# speedup vs baseline: 1.0074x; 1.0074x over previous
import jax
import jax.numpy as jnp
from jax.experimental import pallas as pl
from jax.experimental.pallas import tpu as pltpu

_NUM_MODULES = 11
_HIDDEN = 256
_B = 64
_NEG = -1e30


def _fused_body(xa_ref, xb_ref, xc_ref, xd_ref, w_ref, bp_ref, m_ref, bt_ref, out_ref,
                acc, runmax):
    i = pl.program_id(0)
    nt = pl.num_programs(0)
    t = xa_ref.shape[0]
    w = w_ref[...]

    @pl.when(i == 0)
    def _():
        acc[...] = jnp.zeros((_B, 2 * _NUM_MODULES), jnp.float32)
        runmax[...] = jnp.full((1, _NUM_MODULES), _NEG, jnp.float32)

    for k, xr in enumerate((xa_ref, xb_ref, xc_ref, xd_ref)):
        x = xr[...]
        s = jnp.dot(x, w, preferred_element_type=jnp.float32)
        m = m_ref[k * t:(k + 1) * t].astype(jnp.int32)
        iota = jax.lax.broadcasted_iota(jnp.int32, (1, _NUM_MODULES), 1)
        sa = jnp.where(m == iota, s[:, :_NUM_MODULES], _NEG)
        tmax = jnp.max(sa, axis=0, keepdims=True)

        old_raw = runmax[...]
        new_raw = jnp.maximum(old_raw, tmax)
        runmax[...] = new_raw
        stab_old = jnp.where(old_raw < -1e29, 0.0, old_raw)
        stab_new = jnp.where(new_raw < -1e29, 0.0, new_raw)
        factor = jnp.exp(stab_old - stab_new)

        mex = jnp.exp(sa - stab_new)
        cat = jnp.concatenate(
            [mex, mex * s[:, _NUM_MODULES:2 * _NUM_MODULES]], axis=1)
        bt = bt_ref[0, :, k * t:(k + 1) * t]
        io64 = jax.lax.broadcasted_iota(jnp.int32, (_B, t), 0)
        ohbt = (io64 == bt).astype(jnp.float32)
        contrib = jnp.dot(ohbt, cat, preferred_element_type=jnp.float32)

        facc = jnp.concatenate([factor, factor], axis=1)
        acc[...] = acc[...] * facc + contrib

    @pl.when(i == nt - 1)
    def _():
        den = acc[:, :_NUM_MODULES]
        num = acc[:, _NUM_MODULES:2 * _NUM_MODULES]
        out_ref[...] = (num + bp_ref[...] * den) / (den + 1e-16)


def kernel(x, Wa, ba, Wp, bp, module_assign, batch):
    n = x.shape[0]
    t = 5000
    nt = n // (4 * t)
    wcat = jnp.concatenate([Wa, Wp], axis=0).T
    bp_row = bp[None, :]
    m_col = module_assign.astype(jnp.int8).reshape(n, 1)
    b_row = batch.astype(jnp.int32).reshape(nt, 1, 4 * t)

    out = pl.pallas_call(
        _fused_body,
        grid=(nt,),
        in_specs=[
            pl.BlockSpec((t, _HIDDEN), lambda i: (4 * i, 0)),
            pl.BlockSpec((t, _HIDDEN), lambda i: (4 * i + 1, 0)),
            pl.BlockSpec((t, _HIDDEN), lambda i: (4 * i + 2, 0)),
            pl.BlockSpec((t, _HIDDEN), lambda i: (4 * i + 3, 0)),
            pl.BlockSpec((_HIDDEN, 2 * _NUM_MODULES), lambda i: (0, 0)),
            pl.BlockSpec((1, _NUM_MODULES), lambda i: (0, 0)),
            pl.BlockSpec((4 * t, 1), lambda i: (i, 0)),
            pl.BlockSpec((1, 1, 4 * t), lambda i: (i, 0, 0)),
        ],
        out_specs=pl.BlockSpec((_B, _NUM_MODULES), lambda i: (0, 0)),
        out_shape=jax.ShapeDtypeStruct((_B, _NUM_MODULES), jnp.float32),
        scratch_shapes=[
            pltpu.VMEM((_B, 2 * _NUM_MODULES), jnp.float32),
            pltpu.VMEM((1, _NUM_MODULES), jnp.float32),
        ],
        compiler_params=pltpu.CompilerParams(
            dimension_semantics=("arbitrary",)),
    )(x, x, x, x, wcat, bp_row, m_col, b_row)

    return out


# speedup vs baseline: 1.0428x; 1.0352x over previous
import jax
import jax.numpy as jnp
from jax.experimental import pallas as pl
from jax.experimental.pallas import tpu as pltpu

_NUM_MODULES = 11
_HIDDEN = 256
_B = 64
_NEG = -1e30
_NBUF = 4


def _fused_body(x_hbm, w_ref, bp_ref, m_ref, bt_ref, out_ref,
                b0, b1, b2, b3, sems, acc, runmax):
    i = pl.program_id(0)
    nt = pl.num_programs(0)
    bufs = (b0, b1, b2, b3)
    t = b0.shape[0]
    w = w_ref[...]

    def copy_op(block_idx, k):
        return pltpu.make_async_copy(
            x_hbm.at[pl.ds(block_idx * t, t), :], bufs[k], sems.at[k])

    @pl.when(i == 0)
    def _():
        acc[...] = jnp.zeros((_B, 2 * _NUM_MODULES), jnp.float32)
        runmax[...] = jnp.full((1, _NUM_MODULES), _NEG, jnp.float32)
        for k in range(_NBUF):
            copy_op(k, k).start()

    for k in range(_NBUF):
        copy_op(i * _NBUF + k, k).wait()
        x = bufs[k][...]
        s = jnp.dot(x, w, preferred_element_type=jnp.float32)

        @pl.when(i < nt - 1)
        def _():
            copy_op((i + 1) * _NBUF + k, k).start()

        m = m_ref[k * t:(k + 1) * t].astype(jnp.int32)
        iota = jax.lax.broadcasted_iota(jnp.int32, (1, _NUM_MODULES), 1)
        sa = jnp.where(m == iota, s[:, :_NUM_MODULES], _NEG)
        tmax = jnp.max(sa, axis=0, keepdims=True)

        old_raw = runmax[...]
        new_raw = jnp.maximum(old_raw, tmax)
        runmax[...] = new_raw
        stab_old = jnp.where(old_raw < -1e29, 0.0, old_raw)
        stab_new = jnp.where(new_raw < -1e29, 0.0, new_raw)
        factor = jnp.exp(stab_old - stab_new)

        mex = jnp.exp(sa - stab_new)
        cat = jnp.concatenate(
            [mex, mex * s[:, _NUM_MODULES:2 * _NUM_MODULES]], axis=1)
        bt = bt_ref[0, :, k * t:(k + 1) * t]
        io64 = jax.lax.broadcasted_iota(jnp.int32, (_B, t), 0)
        ohbt = (io64 == bt).astype(jnp.float32)
        contrib = jnp.dot(ohbt, cat, preferred_element_type=jnp.float32)

        facc = jnp.concatenate([factor, factor], axis=1)
        acc[...] = acc[...] * facc + contrib

    @pl.when(i == nt - 1)
    def _():
        den = acc[:, :_NUM_MODULES]
        num = acc[:, _NUM_MODULES:2 * _NUM_MODULES]
        out_ref[...] = (num + bp_ref[...] * den) / (den + 1e-16)


def kernel(x, Wa, ba, Wp, bp, module_assign, batch):
    n = x.shape[0]
    t = 5000
    nt = n // (_NBUF * t)
    wcat = jnp.concatenate([Wa, Wp], axis=0).T
    bp_row = bp[None, :]
    m_col = module_assign.astype(jnp.int8).reshape(n, 1)
    b_row = batch.astype(jnp.int32).reshape(nt, 1, _NBUF * t)

    out = pl.pallas_call(
        _fused_body,
        grid=(nt,),
        in_specs=[
            pl.BlockSpec(memory_space=pl.ANY),
            pl.BlockSpec((_HIDDEN, 2 * _NUM_MODULES), lambda i: (0, 0)),
            pl.BlockSpec((1, _NUM_MODULES), lambda i: (0, 0)),
            pl.BlockSpec((_NBUF * t, 1), lambda i: (i, 0)),
            pl.BlockSpec((1, 1, _NBUF * t), lambda i: (i, 0, 0)),
        ],
        out_specs=pl.BlockSpec((_B, _NUM_MODULES), lambda i: (0, 0)),
        out_shape=jax.ShapeDtypeStruct((_B, _NUM_MODULES), jnp.float32),
        scratch_shapes=[
            pltpu.VMEM((t, _HIDDEN), jnp.float32),
            pltpu.VMEM((t, _HIDDEN), jnp.float32),
            pltpu.VMEM((t, _HIDDEN), jnp.float32),
            pltpu.VMEM((t, _HIDDEN), jnp.float32),
            pltpu.SemaphoreType.DMA((_NBUF,)),
            pltpu.VMEM((_B, 2 * _NUM_MODULES), jnp.float32),
            pltpu.VMEM((1, _NUM_MODULES), jnp.float32),
        ],
        compiler_params=pltpu.CompilerParams(
            dimension_semantics=("arbitrary",)),
    )(x, wcat, bp_row, m_col, b_row)

    return out


# speedup vs baseline: 1.1432x; 1.0962x over previous
import jax
import jax.numpy as jnp
from jax.experimental import pallas as pl
from jax.experimental.pallas import tpu as pltpu

_NUM_MODULES = 11
_HIDDEN = 256
_B = 64
_NEG = -1e30


def _main_body(xa_ref, xb_ref, w_ref, m_ref, bt_ref, pacc_ref, pstab_ref,
               acc, runmax):
    i = pl.program_id(1)
    nt = pl.num_programs(1)
    t = xa_ref.shape[0]
    w = w_ref[...]

    @pl.when(i == 0)
    def _():
        acc[...] = jnp.zeros((_B, 2 * _NUM_MODULES), jnp.float32)
        runmax[...] = jnp.full((1, _NUM_MODULES), _NEG, jnp.float32)

    for k, xr in enumerate((xa_ref, xb_ref)):
        x = xr[...]
        s = jnp.dot(x, w, preferred_element_type=jnp.float32)
        m = m_ref[k * t:(k + 1) * t]
        iota = jax.lax.broadcasted_iota(jnp.int32, (1, _NUM_MODULES), 1)
        sa = jnp.where(m == iota, s[:, :_NUM_MODULES], _NEG)
        tmax = jnp.max(sa, axis=0, keepdims=True)

        old_raw = runmax[...]
        new_raw = jnp.maximum(old_raw, tmax)
        runmax[...] = new_raw
        stab_old = jnp.where(old_raw < -1e29, 0.0, old_raw)
        stab_new = jnp.where(new_raw < -1e29, 0.0, new_raw)
        factor = jnp.exp(stab_old - stab_new)

        mex = jnp.exp(sa - stab_new)
        cat = jnp.concatenate(
            [mex, mex * s[:, _NUM_MODULES:2 * _NUM_MODULES]], axis=1)
        bt = bt_ref[0, :, k * t:(k + 1) * t]
        io64 = jax.lax.broadcasted_iota(jnp.int32, (_B, t), 0)
        ohbt = (io64 == bt).astype(jnp.float32)
        contrib = jnp.dot(ohbt, cat, preferred_element_type=jnp.float32)

        facc = jnp.concatenate([factor, factor], axis=1)
        acc[...] = acc[...] * facc + contrib

    @pl.when(i == nt - 1)
    def _():
        pacc_ref[0] = acc[...]
        pstab_ref[0] = jnp.where(runmax[...] < -1e29, 0.0, runmax[...])


def _combine_body(pacc_ref, pstab_ref, bp_ref, out_ref):
    s0 = pstab_ref[0]
    s1 = pstab_ref[1]
    stab = jnp.maximum(s0, s1)
    f0 = jnp.exp(s0 - stab)
    f1 = jnp.exp(s1 - stab)
    a0 = pacc_ref[0]
    a1 = pacc_ref[1]
    den = a0[:, :_NUM_MODULES] * f0 + a1[:, :_NUM_MODULES] * f1
    num = (a0[:, _NUM_MODULES:2 * _NUM_MODULES] * f0
           + a1[:, _NUM_MODULES:2 * _NUM_MODULES] * f1)
    out_ref[...] = (num + bp_ref[...] * den) / (den + 1e-16)


def kernel(x, Wa, ba, Wp, bp, module_assign, batch):
    n = x.shape[0]
    t = 5000
    nt = n // (2 * 2 * t)
    wcat = jnp.concatenate([Wa, Wp], axis=0).T
    bp_row = bp[None, :]
    m_col = module_assign.astype(jnp.int32).reshape(n, 1)
    b_row = batch.astype(jnp.int32).reshape(2 * nt, 1, 2 * t)

    pacc, pstab = pl.pallas_call(
        _main_body,
        grid=(2, nt),
        in_specs=[
            pl.BlockSpec((t, _HIDDEN), lambda c, i, nt=nt: (2 * (c * nt + i), 0)),
            pl.BlockSpec((t, _HIDDEN), lambda c, i, nt=nt: (2 * (c * nt + i) + 1, 0)),
            pl.BlockSpec((_HIDDEN, 2 * _NUM_MODULES), lambda c, i: (0, 0)),
            pl.BlockSpec((2 * t, 1), lambda c, i, nt=nt: (c * nt + i, 0)),
            pl.BlockSpec((1, 1, 2 * t), lambda c, i, nt=nt: (c * nt + i, 0, 0)),
        ],
        out_specs=[
            pl.BlockSpec((1, _B, 2 * _NUM_MODULES), lambda c, i: (c, 0, 0)),
            pl.BlockSpec((1, 1, _NUM_MODULES), lambda c, i: (c, 0, 0)),
        ],
        out_shape=[
            jax.ShapeDtypeStruct((2, _B, 2 * _NUM_MODULES), jnp.float32),
            jax.ShapeDtypeStruct((2, 1, _NUM_MODULES), jnp.float32),
        ],
        scratch_shapes=[
            pltpu.VMEM((_B, 2 * _NUM_MODULES), jnp.float32),
            pltpu.VMEM((1, _NUM_MODULES), jnp.float32),
        ],
        compiler_params=pltpu.CompilerParams(
            dimension_semantics=("parallel", "arbitrary")),
    )(x, x, wcat, m_col, b_row)

    out = pl.pallas_call(
        _combine_body,
        out_shape=jax.ShapeDtypeStruct((_B, _NUM_MODULES), jnp.float32),
    )(pacc, pstab, bp_row)

    return out


# speedup vs baseline: 1.1848x; 1.0364x over previous
import jax
import jax.numpy as jnp
from jax.experimental import pallas as pl
from jax.experimental.pallas import tpu as pltpu

_NUM_MODULES = 11
_HIDDEN = 256
_B = 64
_NEG = -1e30


def _fused_body(x_ref, w_ref, b_ref, m_ref, bt_ref, out_ref, acc, runmax):
    i = pl.program_id(0)
    nt = pl.num_programs(0)
    x = x_ref[...]
    s = jnp.dot(x, w_ref[...], preferred_element_type=jnp.float32) + b_ref[...]
    m = m_ref[...]
    iota = jax.lax.broadcasted_iota(jnp.int32, (1, _NUM_MODULES), 1)
    oh = (m == iota)
    sa = jnp.where(oh, s[:, :_NUM_MODULES], _NEG)
    tmax = jnp.max(sa, axis=0, keepdims=True)

    @pl.when(i == 0)
    def _():
        acc[...] = jnp.zeros((_B, 2 * _NUM_MODULES), jnp.float32)
        runmax[...] = jnp.full((1, _NUM_MODULES), _NEG, jnp.float32)

    old_raw = runmax[...]
    new_raw = jnp.maximum(old_raw, tmax)
    runmax[...] = new_raw
    stab_old = jnp.where(old_raw < -1e29, 0.0, old_raw)
    stab_new = jnp.where(new_raw < -1e29, 0.0, new_raw)
    factor = jnp.exp(stab_old - stab_new)

    mex = jnp.exp(sa - stab_new)
    mw = mex * s[:, _NUM_MODULES:2 * _NUM_MODULES]
    cat = jnp.concatenate([mex, mw], axis=1)
    bt = bt_ref[0]
    io64 = jax.lax.broadcasted_iota(jnp.int32, (_B, x.shape[0]), 0)
    ohbt = (io64 == bt).astype(jnp.float32)
    contrib = jnp.dot(ohbt, cat, preferred_element_type=jnp.float32)

    facc = jnp.concatenate([factor, factor], axis=1)
    acc[...] = acc[...] * facc + contrib

    @pl.when(i == nt - 1)
    def _():
        out_ref[...] = (acc[:, _NUM_MODULES:2 * _NUM_MODULES]
                        / (acc[:, :_NUM_MODULES] + 1e-16))


def kernel(x, Wa, ba, Wp, bp, module_assign, batch):
    n = x.shape[0]
    t = 10000
    nt = n // t
    wcat = jnp.concatenate([Wa, Wp], axis=0).T
    bcat = jnp.concatenate([ba, bp], axis=0)[None, :]
    m_col = module_assign.astype(jnp.int32).reshape(n, 1)
    b_row = batch.astype(jnp.int32).reshape(nt, 1, t)

    out = pl.pallas_call(
        _fused_body,
        grid=(nt,),
        in_specs=[
            pl.BlockSpec((t, _HIDDEN), lambda i: (i, 0)),
            pl.BlockSpec((_HIDDEN, 2 * _NUM_MODULES), lambda i: (0, 0)),
            pl.BlockSpec((1, 2 * _NUM_MODULES), lambda i: (0, 0)),
            pl.BlockSpec((t, 1), lambda i: (i, 0)),
            pl.BlockSpec((1, 1, t), lambda i: (i, 0, 0)),
        ],
        out_specs=pl.BlockSpec((_B, _NUM_MODULES), lambda i: (0, 0)),
        out_shape=jax.ShapeDtypeStruct((_B, _NUM_MODULES), jnp.float32),
        scratch_shapes=[
            pltpu.VMEM((_B, 2 * _NUM_MODULES), jnp.float32),
            pltpu.VMEM((1, _NUM_MODULES), jnp.float32),
        ],
        compiler_params=pltpu.CompilerParams(
            dimension_semantics=("arbitrary",)),
    )(x, wcat, bcat, m_col, b_row)

    return out
